# baseline (speedup 1.0000x reference)
import jax
import jax.numpy as jnp
from jax import lax
from jax.experimental import pallas as pl
from jax.experimental.pallas import tpu as pltpu
from jax.experimental.pallas import tpu_sc as plsc

T = 1000
D = 128
B = 16384

_info = plsc.get_sparse_core_info()
_NC, _NS = _info.num_cores, _info.num_subcores
_NW = _NC * _NS
_BPW = B // _NW
_NCH = 4
_CHUNK = _BPW // _NCH


def _gather_kernel(table_hbm, t_hbm, out_hbm, idx_v, rows_v, gs0, gs1, ws0, ws1):
    wid = lax.axis_index("s") * _NC + lax.axis_index("c")
    base = wid * _BPW
    gs = (gs0, gs1)
    ws = (ws0, ws1)
    pltpu.sync_copy(t_hbm.at[pl.ds(base, _BPW)], idx_v)

    def start_gather(i):
        return pltpu.async_copy(
            table_hbm.at[idx_v.at[pl.ds(i * _CHUNK, _CHUNK)]],
            rows_v.at[i % 2], gs[i % 2])

    g_cp = [None, None]
    w_cp = [None, None]
    g_cp[0] = start_gather(0)
    for i in range(_NCH):
        nb = (i + 1) % 2
        if i + 1 < _NCH:
            if w_cp[nb] is not None:
                w_cp[nb].wait()
            g_cp[nb] = start_gather(i + 1)
        g_cp[i % 2].wait()
        w_cp[i % 2] = pltpu.async_copy(
            rows_v.at[i % 2],
            out_hbm.at[pl.ds(base + i * _CHUNK, _CHUNK)], ws[i % 2])
    for cp in w_cp:
        if cp is not None:
            cp.wait()


@jax.jit
def _lookup(table, t):
    mesh = plsc.VectorSubcoreMesh(core_axis_name="c", subcore_axis_name="s")
    return pl.kernel(
        _gather_kernel,
        mesh=mesh,
        out_type=jax.ShapeDtypeStruct((B, D), jnp.float32),
        scratch_types=[
            pltpu.VMEM((_BPW,), jnp.int32),
            pltpu.VMEM((2, _CHUNK, D), jnp.float32),
            pltpu.SemaphoreType.DMA,
            pltpu.SemaphoreType.DMA,
            pltpu.SemaphoreType.DMA,
            pltpu.SemaphoreType.DMA,
        ],
    )(table, t)


def kernel(table, t):
    return _lookup(table, t.astype(jnp.int32))


# speedup vs baseline: 1.0168x; 1.0168x over previous
import jax
import jax.numpy as jnp
from jax import lax
from jax.experimental import pallas as pl
from jax.experimental.pallas import tpu as pltpu
from jax.experimental.pallas import tpu_sc as plsc

T = 1000
D = 128
B = 16384

_info = plsc.get_sparse_core_info()
_NC, _NS = _info.num_cores, _info.num_subcores
_NW = _NC * _NS
_BPW = B // _NW
_NCH = 2
_CHUNK = _BPW // _NCH


def _gather_kernel(table_hbm, t_hbm, out_hbm, idx_v, rows_v, gs0, gs1, ws0, ws1):
    wid = lax.axis_index("s") * _NC + lax.axis_index("c")
    base = wid * _BPW
    gs = (gs0, gs1)
    ws = (ws0, ws1)
    pltpu.sync_copy(t_hbm.at[pl.ds(base, _BPW)], idx_v)

    def start_gather(i):
        return pltpu.async_copy(
            table_hbm.at[idx_v.at[pl.ds(i * _CHUNK, _CHUNK)]],
            rows_v.at[i % 2], gs[i % 2])

    g_cp = [None, None]
    w_cp = [None, None]
    g_cp[0] = start_gather(0)
    for i in range(_NCH):
        nb = (i + 1) % 2
        if i + 1 < _NCH:
            if w_cp[nb] is not None:
                w_cp[nb].wait()
            g_cp[nb] = start_gather(i + 1)
        g_cp[i % 2].wait()
        w_cp[i % 2] = pltpu.async_copy(
            rows_v.at[i % 2],
            out_hbm.at[pl.ds(base + i * _CHUNK, _CHUNK)], ws[i % 2])
    for cp in w_cp:
        if cp is not None:
            cp.wait()


@jax.jit
def _lookup(table, t):
    mesh = plsc.VectorSubcoreMesh(core_axis_name="c", subcore_axis_name="s")
    return pl.kernel(
        _gather_kernel,
        mesh=mesh,
        out_type=jax.ShapeDtypeStruct((B, D), jnp.float32),
        scratch_types=[
            pltpu.VMEM((_BPW,), jnp.int32),
            pltpu.VMEM((2, _CHUNK, D), jnp.float32),
            pltpu.SemaphoreType.DMA,
            pltpu.SemaphoreType.DMA,
            pltpu.SemaphoreType.DMA,
            pltpu.SemaphoreType.DMA,
        ],
    )(table, t)


def kernel(table, t):
    return _lookup(table, t.astype(jnp.int32))
